# Initial kernel scaffold; baseline (speedup 1.0000x reference)
import jax
import jax.numpy as jnp
from jax.experimental import pallas as pl


def kernel(x, router_w, expert_w, expert_b, res_w, res_b):
    raise NotImplementedError("write your pallas kernel here")


# speedup vs baseline: 2.5336x; 2.5336x over previous
import functools

import jax
import jax.numpy as jnp
from jax.experimental import pallas as pl

B, CIN, COUT, T, E, K = 2, 768, 1024, 2048, 8, 2
TT = 512
TO = 512


def _body(x_ref, rw_ref, ew_ref, eb_ref, resw_ref, resb_ref,
          out_ref, ti_ref, tv_ref):
    x = x_ref[0]
    logits = jax.lax.dot_general(
        x, rw_ref[...], (((0,), (0,)), ((), ())),
        preferred_element_type=jnp.float32)
    m = jnp.max(logits, axis=-1, keepdims=True)
    p = jnp.exp(logits - m)
    probs = p / jnp.sum(p, axis=-1, keepdims=True)
    eidx = jax.lax.broadcasted_iota(jnp.int32, (TT, E), 1)
    v1 = jnp.max(probs, axis=-1)
    i1 = jnp.min(jnp.where(probs == v1[:, None], eidx, E), axis=-1)
    probs2 = jnp.where(eidx == i1[:, None], -1.0, probs)
    v2 = jnp.max(probs2, axis=-1)
    i2 = jnp.min(jnp.where(probs2 == v2[:, None], eidx, E), axis=-1)
    s = v1 + v2
    g1 = v1 / s
    g2 = v2 / s
    ti_ref[0] = jnp.stack([i1, i2])
    tv_ref[0] = jnp.stack([g1, g2])
    gates = (jnp.where(eidx == i1[:, None], g1[:, None], 0.0)
             + jnp.where(eidx == i2[:, None], g2[:, None], 0.0))

    acc = jax.lax.dot_general(
        resw_ref[...], x, (((1,), (0,)), ((), ())),
        preferred_element_type=jnp.float32)
    acc += resb_ref[0][:, None]
    for e in range(E):
        w = ew_ref[e]
        eo = jax.lax.dot_general(
            w, x, (((1,), (0,)), ((), ())),
            preferred_element_type=jnp.float32)
        eo += eb_ref[e][:, None]
        acc += eo * gates[:, e][None, :]
    out_ref[0] = acc


@jax.jit
def _run(x, router_w, expert_w, expert_b, res_w, res_b):
    grid = (B, T // TT, COUT // TO)
    out, ti, tv = pl.pallas_call(
        _body,
        grid=grid,
        in_specs=[
            pl.BlockSpec((1, CIN, TT), lambda b, t, o: (b, 0, t)),
            pl.BlockSpec((CIN, E), lambda b, t, o: (0, 0)),
            pl.BlockSpec((E, TO, CIN), lambda b, t, o: (0, o, 0)),
            pl.BlockSpec((E, TO), lambda b, t, o: (0, o)),
            pl.BlockSpec((TO, CIN), lambda b, t, o: (o, 0)),
            pl.BlockSpec((1, TO), lambda b, t, o: (0, o)),
        ],
        out_specs=[
            pl.BlockSpec((1, TO, TT), lambda b, t, o: (b, o, t)),
            pl.BlockSpec((1, K, TT), lambda b, t, o: (b, 0, t)),
            pl.BlockSpec((1, K, TT), lambda b, t, o: (b, 0, t)),
        ],
        out_shape=[
            jax.ShapeDtypeStruct((B, COUT, T), jnp.float32),
            jax.ShapeDtypeStruct((B, K, T), jnp.int32),
            jax.ShapeDtypeStruct((B, K, T), jnp.float32),
        ],
    )(x, router_w, expert_w, expert_b, res_w, res_b.reshape(1, COUT))
    topi = jnp.transpose(ti, (0, 2, 1))
    topv = jnp.transpose(tv, (0, 2, 1))
    return out, (topi, topv)


def kernel(x, router_w, expert_w, expert_b, res_w, res_b):
    return _run(x, router_w, expert_w, expert_b, res_w, res_b)


# speedup vs baseline: 2.7124x; 1.0706x over previous
import functools

import jax
import jax.numpy as jnp
from jax.experimental import pallas as pl

B, CIN, COUT, T, E, K = 2, 768, 1024, 2048, 8, 2
TT = 512
TO = 512


def _body(x_ref, rw_ref, ew_ref, eb_ref, resw_ref, resb_ref,
          out_ref, ti_ref, tv_ref):
    x = x_ref[0]
    logits = jax.lax.dot_general(
        x, rw_ref[...], (((0,), (0,)), ((), ())),
        preferred_element_type=jnp.float32)
    m = jnp.max(logits, axis=-1, keepdims=True)
    p = jnp.exp(logits - m)
    probs = p / jnp.sum(p, axis=-1, keepdims=True)
    eidx = jax.lax.broadcasted_iota(jnp.int32, (TT, E), 1)
    v1 = jnp.max(probs, axis=-1)
    i1 = jnp.min(jnp.where(probs == v1[:, None], eidx, E), axis=-1)
    probs2 = jnp.where(eidx == i1[:, None], -1.0, probs)
    v2 = jnp.max(probs2, axis=-1)
    i2 = jnp.min(jnp.where(probs2 == v2[:, None], eidx, E), axis=-1)
    s = v1 + v2
    g1 = v1 / s
    g2 = v2 / s
    ti_ref[0] = jnp.stack([i1, i2])
    tv_ref[0] = jnp.stack([g1, g2])
    gates = (jnp.where(eidx == i1[:, None], g1[:, None], 0.0)
             + jnp.where(eidx == i2[:, None], g2[:, None], 0.0))

    acc = jax.lax.dot_general(
        resw_ref[...], x, (((1,), (0,)), ((), ())),
        preferred_element_type=jnp.float32)
    acc += resb_ref[0][:, None]
    for e in range(E):
        w = ew_ref[e]
        eo = jax.lax.dot_general(
            w, x, (((1,), (0,)), ((), ())),
            preferred_element_type=jnp.float32)
        eo += eb_ref[e][:, None]
        acc += eo * gates[:, e][None, :]
    out_ref[0] = acc


@jax.jit
def _run(x, router_w, expert_w, expert_b, res_w, res_b):
    grid = (COUT // TO, B, T // TT)
    out, ti, tv = pl.pallas_call(
        _body,
        grid=grid,
        in_specs=[
            pl.BlockSpec((1, CIN, TT), lambda o, b, t: (b, 0, t)),
            pl.BlockSpec((CIN, E), lambda o, b, t: (0, 0)),
            pl.BlockSpec((E, TO, CIN), lambda o, b, t: (0, o, 0)),
            pl.BlockSpec((E, TO), lambda o, b, t: (0, o)),
            pl.BlockSpec((TO, CIN), lambda o, b, t: (o, 0)),
            pl.BlockSpec((1, TO), lambda o, b, t: (0, o)),
        ],
        out_specs=[
            pl.BlockSpec((1, TO, TT), lambda o, b, t: (b, o, t)),
            pl.BlockSpec((1, K, TT), lambda o, b, t: (b, 0, t)),
            pl.BlockSpec((1, K, TT), lambda o, b, t: (b, 0, t)),
        ],
        out_shape=[
            jax.ShapeDtypeStruct((B, COUT, T), jnp.float32),
            jax.ShapeDtypeStruct((B, K, T), jnp.int32),
            jax.ShapeDtypeStruct((B, K, T), jnp.float32),
        ],
    )(x, router_w, expert_w, expert_b, res_w, res_b.reshape(1, COUT))
    topi = jnp.transpose(ti, (0, 2, 1))
    topv = jnp.transpose(tv, (0, 2, 1))
    return out, (topi, topv)


def kernel(x, router_w, expert_w, expert_b, res_w, res_b):
    return _run(x, router_w, expert_w, expert_b, res_w, res_b)
